# baseline (speedup 1.0000x reference)
import functools

import jax
import jax.numpy as jnp
import numpy as np
from jax import lax
from jax.experimental import pallas as pl
from jax.experimental.pallas import tpu as pltpu
from jax.experimental.pallas import tpu_sc as plsc

N = 10000
E = 160000
C = 128
EPS = 1e-5

_NC = 2
_NS = 16
_NW = _NC * _NS
_CHUNK = 128
_EPW = 5120
_NCHUNK = _EPW // _CHUNK
E_PAD = _NW * _EPW
_IDXROWS = E_PAD // _CHUNK
N_ACC = 10112
_RPT = N_ACC // _NS
_DUMMY = N


_NSPLIT = 2


def _sc_scatter_body(table, gidx, sidx, zeros_in, out, gidx_v, sidx_v, rows_v,
                     acc, sem0, sem1, sem2, sem3, *, gather):
    cid = lax.axis_index("c")
    sid = lax.axis_index("s")
    wid = sid * _NC + cid
    pltpu.sync_copy(zeros_in.at[pl.ds(sid * _RPT, _RPT)],
                    acc.at[pl.ds(sid * _RPT, _RPT)])
    plsc.subcore_barrier()
    pltpu.sync_copy(sidx.at[pl.ds(wid * _NCHUNK, _NCHUNK)], sidx_v)

    if gather:
        pltpu.sync_copy(gidx.at[pl.ds(wid * _NCHUNK, _NCHUNK)], gidx_v)
        half = _CHUNK // _NSPLIT
        sems = ((sem0, sem1), (sem2, sem3))

        def gcopy(t, b, s):
            return pltpu.make_async_copy(
                table.at[gidx_v.at[t, pl.ds(s * half, half)]],
                rows_v.at[b, pl.ds(s * half, half)], sems[b][s])

        def gissue(t, b):
            for s in range(_NSPLIT):
                gcopy(t, b, s).start()

        gissue(0, 0)

        def body(i, carry):
            for b in range(2):
                t = i * 2 + b
                nxt = t + 1

                @pl.when(nxt < _NCHUNK)
                def _():
                    gissue(nxt, 1 - b)

                for s in range(_NSPLIT):
                    gcopy(t, b, s).wait()
                pltpu.sync_copy(rows_v.at[b], acc.at[sidx_v.at[t]], add=True)
            return carry

        lax.fori_loop(0, _NCHUNK // 2, body, 0)
    else:
        pltpu.sync_copy(table, rows_v.at[0])

        def body(t, carry):
            pltpu.sync_copy(rows_v.at[0], acc.at[sidx_v.at[t]], add=True)
            return carry

        lax.fori_loop(0, _NCHUNK, body, 0)
    plsc.subcore_barrier()
    row0 = sid * _RPT
    off = 0
    while off < _RPT:
        sz = min(_CHUNK, _RPT - off)
        pltpu.sync_copy(acc.at[pl.ds(row0 + off, sz)],
                        rows_v.at[0, pl.ds(0, sz)])
        pltpu.sync_copy(rows_v.at[0, pl.ds(0, sz)],
                        out.at[cid, pl.ds(row0 + off, sz)])
        off += sz


_sc_scatter_cache = {}


def _sc_scatter(*args, gather=True):
    if gather not in _sc_scatter_cache:
        _sc_scatter_cache[gather] = pl.kernel(
            functools.partial(_sc_scatter_body, gather=gather),
            out_type=jax.ShapeDtypeStruct((_NC, N_ACC, C), jnp.float32),
            mesh=plsc.VectorSubcoreMesh(core_axis_name="c",
                                        subcore_axis_name="s"),
            scratch_types=[
                pltpu.VMEM((_NCHUNK, _CHUNK), jnp.int32),
                pltpu.VMEM((_NCHUNK, _CHUNK), jnp.int32),
                pltpu.VMEM((2, _CHUNK, C), jnp.float32),
                pltpu.VMEM_SHARED((N_ACC, C), jnp.float32),
            ] + [pltpu.SemaphoreType.DMA] * 4,
        )
    return _sc_scatter_cache[gather](*args)


_BLK = 1000
_GRID = N // _BLK


def _mm_body(*refs, nx, act, res):
    xs = refs[:nx]
    w = refs[nx][...]
    b = refs[nx + 1][...]
    o = refs[-1]
    x = xs[0][...]
    for xr in xs[1:]:
        x = x + xr[...]
    y = jnp.dot(x, w, preferred_element_type=jnp.float32) + b
    if res:
        y = y + refs[nx + 2][...]
    if act == "relu":
        y = jnp.maximum(y, 0.0)
    o[...] = y


def _mm(xs, w, b, res=None, act=None):
    kin, kout = w.shape
    nx = len(xs)
    ins = list(xs) + [w, b.reshape(1, kout)] + ([res] if res is not None else [])
    specs = ([pl.BlockSpec((_BLK, kin), lambda i: (i, 0))] * nx
             + [pl.BlockSpec((kin, kout), lambda i: (0, 0)),
                pl.BlockSpec((1, kout), lambda i: (0, 0))]
             + ([pl.BlockSpec((_BLK, kout), lambda i: (i, 0))]
                if res is not None else []))
    return pl.pallas_call(
        functools.partial(_mm_body, nx=nx, act=act, res=res is not None),
        grid=(_GRID,),
        in_specs=specs,
        out_specs=pl.BlockSpec((_BLK, kout), lambda i: (i, 0)),
        out_shape=jax.ShapeDtypeStruct((N, kout), jnp.float32),
    )(*ins)


def _mm3_body(x0, x1, x2, w, b, r, o):
    y = jnp.dot(x0[...], w[0], preferred_element_type=jnp.float32)
    y = y + jnp.dot(x1[...], w[1], preferred_element_type=jnp.float32)
    y = y + jnp.dot(x2[...], w[2], preferred_element_type=jnp.float32)
    o[...] = y + b[...] + r[...]


def _mm3(xs, w, b, res):
    return pl.pallas_call(
        _mm3_body,
        grid=(_GRID,),
        in_specs=[pl.BlockSpec((_BLK, C), lambda i: (i, 0))] * 3
        + [pl.BlockSpec((3, C, C), lambda i: (0, 0, 0)),
           pl.BlockSpec((1, C), lambda i: (0, 0)),
           pl.BlockSpec((_BLK, C), lambda i: (i, 0))],
        out_specs=pl.BlockSpec((_BLK, C), lambda i: (i, 0)),
        out_shape=jax.ShapeDtypeStruct((N, C), jnp.float32),
    )(*xs, w, b.reshape(1, C), res)


def _bn_body(*refs, nx, relu):
    h = refs[0][...]
    for r in refs[1:nx]:
        h = h + r[...]
    w = refs[nx][...]
    b = refs[nx + 1][...]
    mu = jnp.mean(h, axis=0, keepdims=True)
    var = jnp.mean((h - mu) ** 2, axis=0, keepdims=True)
    y = (h - mu) * lax.rsqrt(var + EPS) * w + b
    if relu:
        y = jnp.maximum(y, 0.0)
    refs[-1][...] = y


def _bn(xs, w, b, relu=False):
    nx = len(xs)
    ins = list(xs) + [w.reshape(1, C), b.reshape(1, C)]
    return pl.pallas_call(
        functools.partial(_bn_body, nx=nx, relu=relu),
        out_shape=jax.ShapeDtypeStruct((N, C), jnp.float32),
    )(*ins)


def _fa_body(q_ref, k_ref, v_ref, o_ref, *, heads, scale):
    dh = C // heads
    nkb = N // _BLK
    for h in range(heads):
        sl = slice(h * dh, (h + 1) * dh)
        q = q_ref[:, sl] * scale

        def body(i, carry):
            m, l, acc = carry
            kb = k_ref[pl.ds(i * _BLK, _BLK), sl]
            s = lax.dot_general(q, kb, (((1,), (1,)), ((), ())),
                                preferred_element_type=jnp.float32)
            mn = jnp.maximum(m, jnp.max(s, axis=1, keepdims=True))
            p = jnp.exp(s - mn)
            corr = jnp.exp(m - mn)
            vb = v_ref[pl.ds(i * _BLK, _BLK), sl]
            l2 = l * corr + jnp.sum(p, axis=1, keepdims=True)
            acc2 = acc * corr + jnp.dot(p, vb, preferred_element_type=jnp.float32)
            return mn, l2, acc2

        m0 = jnp.full((_BLK, 1), -1e30, jnp.float32)
        l0 = jnp.zeros((_BLK, 1), jnp.float32)
        a0 = jnp.zeros((_BLK, dh), jnp.float32)
        m, l, acc = lax.fori_loop(0, nkb, body, (m0, l0, a0))
        o_ref[:, sl] = acc / l


def _flash(q, k, v, heads):
    scale = 1.0 / np.sqrt(C // heads)
    return pl.pallas_call(
        functools.partial(_fa_body, heads=heads, scale=scale),
        grid=(_GRID,),
        in_specs=[pl.BlockSpec((_BLK, C), lambda i: (i, 0)),
                  pl.BlockSpec((N, C), lambda i: (0, 0)),
                  pl.BlockSpec((N, C), lambda i: (0, 0))],
        out_specs=pl.BlockSpec((_BLK, C), lambda i: (i, 0)),
        out_shape=jax.ShapeDtypeStruct((N, C), jnp.float32),
    )(q, k, v)


def _ew(body, n_out, *xs):
    shape = jax.ShapeDtypeStruct((N, C), jnp.float32)
    return pl.pallas_call(
        body,
        grid=(_GRID,),
        in_specs=[pl.BlockSpec((_BLK, C), lambda i: (i, 0))] * len(xs),
        out_specs=(pl.BlockSpec((_BLK, C), lambda i: (i, 0)),) * n_out
        if n_out > 1 else pl.BlockSpec((_BLK, C), lambda i: (i, 0)),
        out_shape=(shape,) * n_out if n_out > 1 else shape,
    )(*xs)


def _dis_body(d0, d1, o):
    deg = d0[...] + d1[...]
    safe = jnp.where(deg > 0, deg, 1.0)
    o[...] = jnp.where(deg > 0, lax.rsqrt(safe), 0.0)


def _g0_body(x, dis, o):
    o[...] = x[...] * dis[...]


def _tx1_body(p0, p1, dis, tx1, g1):
    d = dis[...]
    t = -d * (p0[...] + p1[...])
    tx1[...] = t
    g1[...] = d * t


def _tx2_body(q0, q1, dis, x, o):
    o[...] = -2.0 * dis[...] * (q0[...] + q1[...]) - x[...]


def _gps_layer(x, dis, gidx, sidx, zeros_acc, p, heads):
    g0 = _ew(_g0_body, 1, x, dis)
    P = _sc_scatter(g0, gidx, sidx, zeros_acc)
    tx1, g1 = _ew(_tx1_body, 2, P[0, :N], P[1, :N], dis)
    Q = _sc_scatter(g1, gidx, sidx, zeros_acc)
    tx2 = _ew(_tx2_body, 1, Q[0, :N], Q[1, :N], dis, x)
    h_local = _bn([_mm3([x, tx1, tx2], p["cheb_w"], p["cheb_b"], res=x)],
                  p["n1_w"], p["n1_b"])
    qkv = _mm([x], p["attn_in_w"].T, p["attn_in_b"])
    attn = _flash(qkv[:, :C], qkv[:, C:2 * C], qkv[:, 2 * C:], heads)
    h_attn = _bn([_mm([attn], p["attn_out_w"].T, p["attn_out_b"], res=x)],
                 p["n2_w"], p["n2_b"])
    m = _mm([h_local, h_attn], p["mlp_w1"].T, p["mlp_b1"], act="relu")
    m2 = _mm([m], p["mlp_w2"].T, p["mlp_b2"])
    return _bn([h_local, h_attn, m2], p["n3_w"], p["n3_b"])


def kernel(x, edge_index, params):
    src = edge_index[0].astype(jnp.int32)
    dst = edge_index[1].astype(jnp.int32)
    pad = E_PAD - E
    padi = lambda a, val: jnp.concatenate(
        [a, jnp.full((pad,), val, jnp.int32)]).reshape(_IDXROWS, _CHUNK)
    zeros_acc = jnp.zeros((N_ACC, C), jnp.float32)
    ones_tab = jnp.ones((_CHUNK, C), jnp.float32)
    gidx = padi(src, 0)
    D = _sc_scatter(ones_tab, gidx, padi(src, _DUMMY), zeros_acc, gather=False)
    dis = _ew(_dis_body, 1, D[0, :N], D[1, :N])
    sidx = padi(dst, _DUMMY)

    h = _gps_layer(x, dis, gidx, sidx, zeros_acc, params["gps1"], 2)
    h = _bn([h], params["bn1_w"], params["bn1_b"], relu=True)
    h = _gps_layer(h, dis, gidx, sidx, zeros_acc, params["gps2"], 1)
    h = _bn([h], params["bn2_w"], params["bn2_b"], relu=True)
    return _mm([h], params["lin1_w"].T, params["lin1_b"])


# speedup vs baseline: 1.0003x; 1.0003x over previous
import functools

import jax
import jax.numpy as jnp
import numpy as np
from jax import lax
from jax.experimental import pallas as pl
from jax.experimental.pallas import tpu as pltpu
from jax.experimental.pallas import tpu_sc as plsc

N = 10000
E = 160000
C = 128
EPS = 1e-5

_NC = 2
_NS = 16
_NW = _NC * _NS
_CHUNK = 128
_EPW = 5120
_NCHUNK = _EPW // _CHUNK
E_PAD = _NW * _EPW
_IDXROWS = E_PAD // _CHUNK
N_ACC = 10112
_RPT = N_ACC // _NS
_DUMMY = N


_NSPLIT = 2


def _sc_scatter_body(table, gidx, sidx, zeros_in, out, gidx_v, sidx_v, rows_v,
                     acc, sem0, sem1, sem2, sem3, *, gather):
    cid = lax.axis_index("c")
    sid = lax.axis_index("s")
    wid = sid * _NC + cid
    pltpu.sync_copy(zeros_in.at[pl.ds(sid * _RPT, _RPT)],
                    acc.at[pl.ds(sid * _RPT, _RPT)])
    plsc.subcore_barrier()
    pltpu.sync_copy(sidx.at[pl.ds(wid * _NCHUNK, _NCHUNK)], sidx_v)

    if gather:
        pltpu.sync_copy(gidx.at[pl.ds(wid * _NCHUNK, _NCHUNK)], gidx_v)
        sems = (sem0, sem1)
        pltpu.async_copy(table.at[gidx_v.at[0]], rows_v.at[0], sem0)

        def body(i, carry):
            for b in range(2):
                t = i * 2 + b
                nxt = t + 1

                @pl.when(nxt < _NCHUNK)
                def _():
                    pltpu.async_copy(table.at[gidx_v.at[nxt]],
                                     rows_v.at[1 - b], sems[1 - b])

                pltpu.make_async_copy(table.at[gidx_v.at[t]], rows_v.at[b],
                                      sems[b]).wait()
                pltpu.sync_copy(rows_v.at[b], acc.at[sidx_v.at[t]], add=True)
            return carry

        lax.fori_loop(0, _NCHUNK // 2, body, 0)
    else:
        pltpu.sync_copy(table, rows_v.at[0])

        def body(t, carry):
            pltpu.sync_copy(rows_v.at[0], acc.at[sidx_v.at[t]], add=True)
            return carry

        lax.fori_loop(0, _NCHUNK, body, 0)
    plsc.subcore_barrier()
    row0 = sid * _RPT
    off = 0
    while off < _RPT:
        sz = min(_CHUNK, _RPT - off)
        pltpu.sync_copy(acc.at[pl.ds(row0 + off, sz)],
                        rows_v.at[0, pl.ds(0, sz)])
        pltpu.sync_copy(rows_v.at[0, pl.ds(0, sz)],
                        out.at[cid, pl.ds(row0 + off, sz)])
        off += sz


_sc_scatter_cache = {}


def _sc_scatter(*args, gather=True):
    if gather not in _sc_scatter_cache:
        _sc_scatter_cache[gather] = pl.kernel(
            functools.partial(_sc_scatter_body, gather=gather),
            out_type=jax.ShapeDtypeStruct((_NC, N_ACC, C), jnp.float32),
            mesh=plsc.VectorSubcoreMesh(core_axis_name="c",
                                        subcore_axis_name="s"),
            scratch_types=[
                pltpu.VMEM((_NCHUNK, _CHUNK), jnp.int32),
                pltpu.VMEM((_NCHUNK, _CHUNK), jnp.int32),
                pltpu.VMEM((2, _CHUNK, C), jnp.float32),
                pltpu.VMEM_SHARED((N_ACC, C), jnp.float32),
            ] + [pltpu.SemaphoreType.DMA] * 4,
        )
    return _sc_scatter_cache[gather](*args)


_BLK = 1000
_GRID = N // _BLK


def _mm_body(*refs, nx, act, res):
    xs = refs[:nx]
    w = refs[nx][...]
    b = refs[nx + 1][...]
    o = refs[-1]
    x = xs[0][...]
    for xr in xs[1:]:
        x = x + xr[...]
    y = jnp.dot(x, w, preferred_element_type=jnp.float32) + b
    if res:
        y = y + refs[nx + 2][...]
    if act == "relu":
        y = jnp.maximum(y, 0.0)
    o[...] = y


def _mm(xs, w, b, res=None, act=None):
    kin, kout = w.shape
    nx = len(xs)
    ins = list(xs) + [w, b.reshape(1, kout)] + ([res] if res is not None else [])
    specs = ([pl.BlockSpec((_BLK, kin), lambda i: (i, 0))] * nx
             + [pl.BlockSpec((kin, kout), lambda i: (0, 0)),
                pl.BlockSpec((1, kout), lambda i: (0, 0))]
             + ([pl.BlockSpec((_BLK, kout), lambda i: (i, 0))]
                if res is not None else []))
    return pl.pallas_call(
        functools.partial(_mm_body, nx=nx, act=act, res=res is not None),
        grid=(_GRID,),
        in_specs=specs,
        out_specs=pl.BlockSpec((_BLK, kout), lambda i: (i, 0)),
        out_shape=jax.ShapeDtypeStruct((N, kout), jnp.float32),
    )(*ins)


def _mm3_body(x0, x1, x2, w, b, r, o):
    y = jnp.dot(x0[...], w[0], preferred_element_type=jnp.float32)
    y = y + jnp.dot(x1[...], w[1], preferred_element_type=jnp.float32)
    y = y + jnp.dot(x2[...], w[2], preferred_element_type=jnp.float32)
    o[...] = y + b[...] + r[...]


def _mm3(xs, w, b, res):
    return pl.pallas_call(
        _mm3_body,
        grid=(_GRID,),
        in_specs=[pl.BlockSpec((_BLK, C), lambda i: (i, 0))] * 3
        + [pl.BlockSpec((3, C, C), lambda i: (0, 0, 0)),
           pl.BlockSpec((1, C), lambda i: (0, 0)),
           pl.BlockSpec((_BLK, C), lambda i: (i, 0))],
        out_specs=pl.BlockSpec((_BLK, C), lambda i: (i, 0)),
        out_shape=jax.ShapeDtypeStruct((N, C), jnp.float32),
    )(*xs, w, b.reshape(1, C), res)


def _bn_body(*refs, nx, relu):
    h = refs[0][...]
    for r in refs[1:nx]:
        h = h + r[...]
    w = refs[nx][...]
    b = refs[nx + 1][...]
    mu = jnp.mean(h, axis=0, keepdims=True)
    var = jnp.mean((h - mu) ** 2, axis=0, keepdims=True)
    y = (h - mu) * lax.rsqrt(var + EPS) * w + b
    if relu:
        y = jnp.maximum(y, 0.0)
    refs[-1][...] = y


def _bn(xs, w, b, relu=False):
    nx = len(xs)
    ins = list(xs) + [w.reshape(1, C), b.reshape(1, C)]
    return pl.pallas_call(
        functools.partial(_bn_body, nx=nx, relu=relu),
        out_shape=jax.ShapeDtypeStruct((N, C), jnp.float32),
    )(*ins)


def _fa_body(q_ref, k_ref, v_ref, o_ref, *, heads, scale):
    dh = C // heads
    nkb = N // _BLK
    for h in range(heads):
        sl = slice(h * dh, (h + 1) * dh)
        q = q_ref[:, sl] * scale

        def body(i, carry):
            m, l, acc = carry
            kb = k_ref[pl.ds(i * _BLK, _BLK), sl]
            s = lax.dot_general(q, kb, (((1,), (1,)), ((), ())),
                                preferred_element_type=jnp.float32)
            mn = jnp.maximum(m, jnp.max(s, axis=1, keepdims=True))
            p = jnp.exp(s - mn)
            corr = jnp.exp(m - mn)
            vb = v_ref[pl.ds(i * _BLK, _BLK), sl]
            l2 = l * corr + jnp.sum(p, axis=1, keepdims=True)
            acc2 = acc * corr + jnp.dot(p, vb, preferred_element_type=jnp.float32)
            return mn, l2, acc2

        m0 = jnp.full((_BLK, 1), -1e30, jnp.float32)
        l0 = jnp.zeros((_BLK, 1), jnp.float32)
        a0 = jnp.zeros((_BLK, dh), jnp.float32)
        m, l, acc = lax.fori_loop(0, nkb, body, (m0, l0, a0))
        o_ref[:, sl] = acc / l


def _flash(q, k, v, heads):
    scale = 1.0 / np.sqrt(C // heads)
    return pl.pallas_call(
        functools.partial(_fa_body, heads=heads, scale=scale),
        grid=(_GRID,),
        in_specs=[pl.BlockSpec((_BLK, C), lambda i: (i, 0)),
                  pl.BlockSpec((N, C), lambda i: (0, 0)),
                  pl.BlockSpec((N, C), lambda i: (0, 0))],
        out_specs=pl.BlockSpec((_BLK, C), lambda i: (i, 0)),
        out_shape=jax.ShapeDtypeStruct((N, C), jnp.float32),
    )(q, k, v)


def _ew(body, n_out, *xs):
    shape = jax.ShapeDtypeStruct((N, C), jnp.float32)
    return pl.pallas_call(
        body,
        grid=(_GRID,),
        in_specs=[pl.BlockSpec((_BLK, C), lambda i: (i, 0))] * len(xs),
        out_specs=(pl.BlockSpec((_BLK, C), lambda i: (i, 0)),) * n_out
        if n_out > 1 else pl.BlockSpec((_BLK, C), lambda i: (i, 0)),
        out_shape=(shape,) * n_out if n_out > 1 else shape,
    )(*xs)


def _dis_body(d0, d1, o):
    deg = d0[...] + d1[...]
    safe = jnp.where(deg > 0, deg, 1.0)
    o[...] = jnp.where(deg > 0, lax.rsqrt(safe), 0.0)


def _g0_body(x, dis, o):
    o[...] = x[...] * dis[...]


def _tx1_body(p0, p1, dis, tx1, g1):
    d = dis[...]
    t = -d * (p0[...] + p1[...])
    tx1[...] = t
    g1[...] = d * t


def _tx2_body(q0, q1, dis, x, o):
    o[...] = -2.0 * dis[...] * (q0[...] + q1[...]) - x[...]


def _gps_layer(x, dis, gidx, sidx, zeros_acc, p, heads):
    g0 = _ew(_g0_body, 1, x, dis)
    P = _sc_scatter(g0, gidx, sidx, zeros_acc)
    tx1, g1 = _ew(_tx1_body, 2, P[0, :N], P[1, :N], dis)
    Q = _sc_scatter(g1, gidx, sidx, zeros_acc)
    tx2 = _ew(_tx2_body, 1, Q[0, :N], Q[1, :N], dis, x)
    h_local = _bn([_mm3([x, tx1, tx2], p["cheb_w"], p["cheb_b"], res=x)],
                  p["n1_w"], p["n1_b"])
    qkv = _mm([x], p["attn_in_w"].T, p["attn_in_b"])
    attn = _flash(qkv[:, :C], qkv[:, C:2 * C], qkv[:, 2 * C:], heads)
    h_attn = _bn([_mm([attn], p["attn_out_w"].T, p["attn_out_b"], res=x)],
                 p["n2_w"], p["n2_b"])
    m = _mm([h_local, h_attn], p["mlp_w1"].T, p["mlp_b1"], act="relu")
    m2 = _mm([m], p["mlp_w2"].T, p["mlp_b2"])
    return _bn([h_local, h_attn, m2], p["n3_w"], p["n3_b"])


def kernel(x, edge_index, params):
    src = edge_index[0].astype(jnp.int32)
    dst = edge_index[1].astype(jnp.int32)
    pad = E_PAD - E
    padi = lambda a, val: jnp.concatenate(
        [a, jnp.full((pad,), val, jnp.int32)]).reshape(_IDXROWS, _CHUNK)
    zeros_acc = jnp.zeros((N_ACC, C), jnp.float32)
    ones_tab = jnp.ones((_CHUNK, C), jnp.float32)
    gidx = padi(src, 0)
    D = _sc_scatter(ones_tab, gidx, padi(src, _DUMMY), zeros_acc, gather=False)
    dis = _ew(_dis_body, 1, D[0, :N], D[1, :N])
    sidx = padi(dst, _DUMMY)

    h = _gps_layer(x, dis, gidx, sidx, zeros_acc, params["gps1"], 2)
    h = _bn([h], params["bn1_w"], params["bn1_b"], relu=True)
    h = _gps_layer(h, dis, gidx, sidx, zeros_acc, params["gps2"], 1)
    h = _bn([h], params["bn2_w"], params["bn2_b"], relu=True)
    return _mm([h], params["lin1_w"].T, params["lin1_b"])


# speedup vs baseline: 1.0255x; 1.0251x over previous
import functools

import jax
import jax.numpy as jnp
import numpy as np
from jax import lax
from jax.experimental import pallas as pl
from jax.experimental.pallas import tpu as pltpu
from jax.experimental.pallas import tpu_sc as plsc

N = 10000
E = 160000
C = 128
EPS = 1e-5

_NC = 2
_NS = 16
_NW = _NC * _NS
_CHUNK = 128
_EPW = 5120
_NCHUNK = _EPW // _CHUNK
E_PAD = _NW * _EPW
_CH0 = 64
_CH1 = 2 * _NCHUNK - _CH0
_IDXROWS = E_PAD // _CHUNK + (_CH0 - _CH1)
N_ACC = 10112
_RPT = N_ACC // _NS
_DUMMY = N


_NSPLIT = 2


def _sc_scatter_body(table, gidx, sidx, zeros_in, out, gidx_v, sidx_v, rows_v,
                     acc, sem0, sem1, sem2, sem3, *, gather):
    cid = lax.axis_index("c")
    sid = lax.axis_index("s")
    nch = jnp.where(cid == 0, _CH0, _CH1)
    base = jnp.where(cid == 0, sid * _CH0, _NS * _CH0 + sid * _CH1)
    pltpu.sync_copy(zeros_in.at[pl.ds(sid * _RPT, _RPT)],
                    acc.at[pl.ds(sid * _RPT, _RPT)])
    plsc.subcore_barrier()
    pltpu.sync_copy(sidx.at[pl.ds(base, _CH0)], sidx_v)

    if gather:
        pltpu.sync_copy(gidx.at[pl.ds(base, _CH0)], gidx_v)
        sems = (sem0, sem1)
        pltpu.async_copy(table.at[gidx_v.at[0]], rows_v.at[0], sem0)

        def body(i, carry):
            for b in range(2):
                t = i * 2 + b
                nxt = t + 1

                @pl.when(nxt < nch)
                def _():
                    pltpu.async_copy(table.at[gidx_v.at[nxt]],
                                     rows_v.at[1 - b], sems[1 - b])

                pltpu.make_async_copy(table.at[gidx_v.at[t]], rows_v.at[b],
                                      sems[b]).wait()
                pltpu.sync_copy(rows_v.at[b], acc.at[sidx_v.at[t]], add=True)
            return carry

        lax.fori_loop(0, nch // 2, body, 0)
    else:
        pltpu.sync_copy(table, rows_v.at[0])

        def body(t, carry):
            pltpu.sync_copy(rows_v.at[0], acc.at[sidx_v.at[t]], add=True)
            return carry

        lax.fori_loop(0, nch, body, 0)
    plsc.subcore_barrier()
    row0 = sid * _RPT
    off = 0
    while off < _RPT:
        sz = min(_CHUNK, _RPT - off)
        pltpu.sync_copy(acc.at[pl.ds(row0 + off, sz)],
                        rows_v.at[0, pl.ds(0, sz)])
        pltpu.sync_copy(rows_v.at[0, pl.ds(0, sz)],
                        out.at[cid, pl.ds(row0 + off, sz)])
        off += sz


_sc_scatter_cache = {}


def _sc_scatter(*args, gather=True):
    if gather not in _sc_scatter_cache:
        _sc_scatter_cache[gather] = pl.kernel(
            functools.partial(_sc_scatter_body, gather=gather),
            out_type=jax.ShapeDtypeStruct((_NC, N_ACC, C), jnp.float32),
            mesh=plsc.VectorSubcoreMesh(core_axis_name="c",
                                        subcore_axis_name="s"),
            scratch_types=[
                pltpu.VMEM((_CH0, _CHUNK), jnp.int32),
                pltpu.VMEM((_CH0, _CHUNK), jnp.int32),
                pltpu.VMEM((2, _CHUNK, C), jnp.float32),
                pltpu.VMEM_SHARED((N_ACC, C), jnp.float32),
            ] + [pltpu.SemaphoreType.DMA] * 4,
        )
    return _sc_scatter_cache[gather](*args)


_BLK = 1000
_GRID = N // _BLK


def _mm_body(*refs, nx, act, res):
    xs = refs[:nx]
    w = refs[nx][...]
    b = refs[nx + 1][...]
    o = refs[-1]
    x = xs[0][...]
    for xr in xs[1:]:
        x = x + xr[...]
    y = jnp.dot(x, w, preferred_element_type=jnp.float32) + b
    if res:
        y = y + refs[nx + 2][...]
    if act == "relu":
        y = jnp.maximum(y, 0.0)
    o[...] = y


def _mm(xs, w, b, res=None, act=None):
    kin, kout = w.shape
    nx = len(xs)
    ins = list(xs) + [w, b.reshape(1, kout)] + ([res] if res is not None else [])
    specs = ([pl.BlockSpec((_BLK, kin), lambda i: (i, 0))] * nx
             + [pl.BlockSpec((kin, kout), lambda i: (0, 0)),
                pl.BlockSpec((1, kout), lambda i: (0, 0))]
             + ([pl.BlockSpec((_BLK, kout), lambda i: (i, 0))]
                if res is not None else []))
    return pl.pallas_call(
        functools.partial(_mm_body, nx=nx, act=act, res=res is not None),
        grid=(_GRID,),
        in_specs=specs,
        out_specs=pl.BlockSpec((_BLK, kout), lambda i: (i, 0)),
        out_shape=jax.ShapeDtypeStruct((N, kout), jnp.float32),
    )(*ins)


def _mm3_body(x0, x1, x2, w, b, r, o):
    y = jnp.dot(x0[...], w[0], preferred_element_type=jnp.float32)
    y = y + jnp.dot(x1[...], w[1], preferred_element_type=jnp.float32)
    y = y + jnp.dot(x2[...], w[2], preferred_element_type=jnp.float32)
    o[...] = y + b[...] + r[...]


def _mm3(xs, w, b, res):
    return pl.pallas_call(
        _mm3_body,
        grid=(_GRID,),
        in_specs=[pl.BlockSpec((_BLK, C), lambda i: (i, 0))] * 3
        + [pl.BlockSpec((3, C, C), lambda i: (0, 0, 0)),
           pl.BlockSpec((1, C), lambda i: (0, 0)),
           pl.BlockSpec((_BLK, C), lambda i: (i, 0))],
        out_specs=pl.BlockSpec((_BLK, C), lambda i: (i, 0)),
        out_shape=jax.ShapeDtypeStruct((N, C), jnp.float32),
    )(*xs, w, b.reshape(1, C), res)


def _bn_body(*refs, nx, relu):
    h = refs[0][...]
    for r in refs[1:nx]:
        h = h + r[...]
    w = refs[nx][...]
    b = refs[nx + 1][...]
    mu = jnp.mean(h, axis=0, keepdims=True)
    var = jnp.mean((h - mu) ** 2, axis=0, keepdims=True)
    y = (h - mu) * lax.rsqrt(var + EPS) * w + b
    if relu:
        y = jnp.maximum(y, 0.0)
    refs[-1][...] = y


def _bn(xs, w, b, relu=False):
    nx = len(xs)
    ins = list(xs) + [w.reshape(1, C), b.reshape(1, C)]
    return pl.pallas_call(
        functools.partial(_bn_body, nx=nx, relu=relu),
        out_shape=jax.ShapeDtypeStruct((N, C), jnp.float32),
    )(*ins)


def _fa_body(q_ref, k_ref, v_ref, o_ref, *, heads, scale):
    dh = C // heads
    nkb = N // _BLK
    for h in range(heads):
        sl = slice(h * dh, (h + 1) * dh)
        q = q_ref[:, sl] * scale

        def body(i, carry):
            m, l, acc = carry
            kb = k_ref[pl.ds(i * _BLK, _BLK), sl]
            s = lax.dot_general(q, kb, (((1,), (1,)), ((), ())),
                                preferred_element_type=jnp.float32)
            mn = jnp.maximum(m, jnp.max(s, axis=1, keepdims=True))
            p = jnp.exp(s - mn)
            corr = jnp.exp(m - mn)
            vb = v_ref[pl.ds(i * _BLK, _BLK), sl]
            l2 = l * corr + jnp.sum(p, axis=1, keepdims=True)
            acc2 = acc * corr + jnp.dot(p, vb, preferred_element_type=jnp.float32)
            return mn, l2, acc2

        m0 = jnp.full((_BLK, 1), -1e30, jnp.float32)
        l0 = jnp.zeros((_BLK, 1), jnp.float32)
        a0 = jnp.zeros((_BLK, dh), jnp.float32)
        m, l, acc = lax.fori_loop(0, nkb, body, (m0, l0, a0))
        o_ref[:, sl] = acc / l


def _flash(q, k, v, heads):
    scale = 1.0 / np.sqrt(C // heads)
    return pl.pallas_call(
        functools.partial(_fa_body, heads=heads, scale=scale),
        grid=(_GRID,),
        in_specs=[pl.BlockSpec((_BLK, C), lambda i: (i, 0)),
                  pl.BlockSpec((N, C), lambda i: (0, 0)),
                  pl.BlockSpec((N, C), lambda i: (0, 0))],
        out_specs=pl.BlockSpec((_BLK, C), lambda i: (i, 0)),
        out_shape=jax.ShapeDtypeStruct((N, C), jnp.float32),
    )(q, k, v)


def _ew(body, n_out, *xs):
    shape = jax.ShapeDtypeStruct((N, C), jnp.float32)
    return pl.pallas_call(
        body,
        grid=(_GRID,),
        in_specs=[pl.BlockSpec((_BLK, C), lambda i: (i, 0))] * len(xs),
        out_specs=(pl.BlockSpec((_BLK, C), lambda i: (i, 0)),) * n_out
        if n_out > 1 else pl.BlockSpec((_BLK, C), lambda i: (i, 0)),
        out_shape=(shape,) * n_out if n_out > 1 else shape,
    )(*xs)


def _dis_body(d0, d1, o):
    deg = d0[...] + d1[...]
    safe = jnp.where(deg > 0, deg, 1.0)
    o[...] = jnp.where(deg > 0, lax.rsqrt(safe), 0.0)


def _g0_body(x, dis, o):
    o[...] = x[...] * dis[...]


def _tx1_body(p0, p1, dis, tx1, g1):
    d = dis[...]
    t = -d * (p0[...] + p1[...])
    tx1[...] = t
    g1[...] = d * t


def _tx2_body(q0, q1, dis, x, o):
    o[...] = -2.0 * dis[...] * (q0[...] + q1[...]) - x[...]


def _gps_layer(x, dis, gidx, sidx, zeros_acc, p, heads):
    g0 = _ew(_g0_body, 1, x, dis)
    P = _sc_scatter(g0, gidx, sidx, zeros_acc)
    tx1, g1 = _ew(_tx1_body, 2, P[0, :N], P[1, :N], dis)
    Q = _sc_scatter(g1, gidx, sidx, zeros_acc)
    tx2 = _ew(_tx2_body, 1, Q[0, :N], Q[1, :N], dis, x)
    h_local = _bn([_mm3([x, tx1, tx2], p["cheb_w"], p["cheb_b"], res=x)],
                  p["n1_w"], p["n1_b"])
    qkv = _mm([x], p["attn_in_w"].T, p["attn_in_b"])
    attn = _flash(qkv[:, :C], qkv[:, C:2 * C], qkv[:, 2 * C:], heads)
    h_attn = _bn([_mm([attn], p["attn_out_w"].T, p["attn_out_b"], res=x)],
                 p["n2_w"], p["n2_b"])
    m = _mm([h_local, h_attn], p["mlp_w1"].T, p["mlp_b1"], act="relu")
    m2 = _mm([m], p["mlp_w2"].T, p["mlp_b2"])
    return _bn([h_local, h_attn, m2], p["n3_w"], p["n3_b"])


def kernel(x, edge_index, params):
    src = edge_index[0].astype(jnp.int32)
    dst = edge_index[1].astype(jnp.int32)
    pad = _IDXROWS * _CHUNK - E
    padi = lambda a, val: jnp.concatenate(
        [a, jnp.full((pad,), val, jnp.int32)]).reshape(_IDXROWS, _CHUNK)
    zeros_acc = jnp.zeros((N_ACC, C), jnp.float32)
    ones_tab = jnp.ones((_CHUNK, C), jnp.float32)
    gidx = padi(src, 0)
    D = _sc_scatter(ones_tab, gidx, padi(src, _DUMMY), zeros_acc, gather=False)
    dis = _ew(_dis_body, 1, D[0, :N], D[1, :N])
    sidx = padi(dst, _DUMMY)

    h = _gps_layer(x, dis, gidx, sidx, zeros_acc, params["gps1"], 2)
    h = _bn([h], params["bn1_w"], params["bn1_b"], relu=True)
    h = _gps_layer(h, dis, gidx, sidx, zeros_acc, params["gps2"], 1)
    h = _bn([h], params["bn2_w"], params["bn2_b"], relu=True)
    return _mm([h], params["lin1_w"].T, params["lin1_b"])
